# baseline (speedup 1.0000x reference)
import functools

import jax
import jax.numpy as jnp
from jax import lax
from jax.experimental import pallas as pl
from jax.experimental.pallas import tpu as pltpu
from jax.experimental.pallas import tpu_sc as plsc

NP_ = 32
NK_ = 32
M_ = NP_ * NK_
EMBED_ = 256
BR_ = 2048


def _tc_body(q_ref, P_ref, pred_ref, D_ref):
    q = q_ref[...]
    Pm = P_ref[...]
    qb = (q * -2.0).astype(jnp.bfloat16)
    Pb = Pm.astype(jnp.bfloat16)
    S = jax.lax.dot_general(
        qb, Pb, (((1,), (1,)), ((), ())),
        preferred_element_type=jnp.float32)
    q2 = jnp.sum(q * q, axis=1, keepdims=True)
    p2 = jnp.sum(Pm * Pm, axis=1)[None, :]
    m = jnp.maximum((q2 + p2) + S, 1e-12)
    D = m * jax.lax.rsqrt(m)
    E = jnp.exp(-D)
    D_ref[...] = D
    col = jax.lax.broadcasted_iota(jnp.int32, (M_, NP_), 0)
    grp = jax.lax.broadcasted_iota(jnp.int32, (M_, NP_), 1)
    G = jnp.where(col // NK_ == grp, 1.0, 0.0).astype(jnp.float32)
    ps = jax.lax.dot_general(
        E, G, (((1,), (0,)), ((), ())),
        preferred_element_type=jnp.float32)
    pred_ref[...] = ps / jnp.sum(ps, axis=1, keepdims=True)


def _tc_call(q, P, chunk, nrows):
    nb = nrows // BR_
    base = chunk * nb
    return pl.pallas_call(
        _tc_body,
        grid=(nb,),
        in_specs=[
            pl.BlockSpec((BR_, EMBED_), lambda i: (base + i, 0)),
            pl.BlockSpec((M_, EMBED_), lambda i: (0, 0)),
        ],
        out_specs=[
            pl.BlockSpec((BR_, NP_), lambda i: (i, 0)),
            pl.BlockSpec((BR_, M_), lambda i: (i, 0)),
        ],
        out_shape=[
            jax.ShapeDtypeStruct((nrows, NP_), jnp.float32),
            jax.ShapeDtypeStruct((nrows, M_), jnp.float32),
        ],
        compiler_params=pltpu.CompilerParams(
            dimension_semantics=("arbitrary",)),
    )(q, P)


_NC = 2
_NS = 16
_NW = _NC * _NS
_LANES = 16
_CH = 32768


_NBUF = 3
_CHR = 32


def _make_sc_exp(B):
    rows_w = B // _NW
    nch = rows_w // _CHR
    mesh = plsc.VectorSubcoreMesh(core_axis_name="c", subcore_axis_name="s")

    @functools.partial(
        pl.kernel, mesh=mesh,
        out_type=jax.ShapeDtypeStruct((B, M_), jnp.float32),
        scratch_types=(
            [pltpu.VMEM((_CHR, M_), jnp.float32) for _ in range(_NBUF)]
            + [pltpu.SemaphoreType.DMA for _ in range(2 * _NBUF)]
        ),
        compiler_params=pltpu.CompilerParams(use_tc_tiling_on_sc=True),
    )
    def sc_exp(d_hbm, e_hbm, *scratch):
        bufs = scratch[:_NBUF]
        sem_in = scratch[_NBUF:2 * _NBUF]
        sem_out = scratch[2 * _NBUF:]
        wid = lax.axis_index("s") * _NC + lax.axis_index("c")
        base = pl.multiple_of(wid * rows_w, _CHR)

        def start_in(c):
            return pltpu.async_copy(
                d_hbm.at[pl.ds(base + c * _CHR, _CHR), :], bufs[c % _NBUF],
                sem_in[c % _NBUF])

        def start_out(c):
            return pltpu.async_copy(
                bufs[c % _NBUF], e_hbm.at[pl.ds(base + c * _CHR, _CHR), :],
                sem_out[c % _NBUF])

        def compute(buf):
            def body(i, carry):
                for k in range(M_ // _LANES):
                    sl = pl.ds(k * _LANES, _LANES)
                    buf[i, sl] = jnp.exp(-buf[i, sl])
                return carry

            lax.fori_loop(0, _CHR, body, 0)

        h_in = {}
        h_out = {}
        h_in[0] = start_in(0)
        for c in range(nch):
            if c + 1 < nch:
                if c + 1 >= _NBUF:
                    h_out[c + 1 - _NBUF].wait()
                h_in[c + 1] = start_in(c + 1)
            h_in[c].wait()
            compute(bufs[c % _NBUF])
            h_out[c] = start_out(c)
        for c in range(max(0, nch - _NBUF), nch):
            h_out[c].wait()

    return sc_exp


@functools.partial(jax.jit, static_argnames=())
def kernel(q, P):
    B = q.shape[0]
    nchunk = 4
    nrows = B // nchunk
    sc_exp = _make_sc_exp(nrows)
    preds, Ds, Es = [], [], []
    for c in range(nchunk):
        pred_c, D_c = _tc_call(q, P, c, nrows)
        preds.append(pred_c)
        Ds.append(D_c)
        Es.append(sc_exp(D_c))
    pred = jnp.concatenate(preds, axis=0)
    D = jnp.concatenate(Ds, axis=0)
    E = jnp.concatenate(Es, axis=0)
    return (pred, D, E.reshape(B, NP_, NK_))


# speedup vs baseline: 2.3589x; 2.3589x over previous
import functools

import jax
import jax.numpy as jnp
from jax.experimental import pallas as pl
from jax.experimental.pallas import tpu as pltpu

NP_ = 32
NK_ = 32
M_ = NP_ * NK_
EMBED_ = 256
BR_ = 1024


def _body(q_ref, P_ref, pred_ref, D_ref, E_ref):
    q = q_ref[...]
    Pm = P_ref[...]
    qb = (q * -2.0).astype(jnp.bfloat16)
    Pb = Pm.astype(jnp.bfloat16)
    S = jax.lax.dot_general(
        qb, Pb, (((1,), (1,)), ((), ())),
        preferred_element_type=jnp.float32)
    q2 = jnp.sum(q * q, axis=1, keepdims=True)
    p2 = jnp.sum(Pm * Pm, axis=1)[None, :]
    m = jnp.maximum((q2 + p2) + S, 1e-12)
    D = m * jax.lax.rsqrt(m)
    E = jnp.exp(-D)
    D_ref[...] = D
    E_ref[...] = E
    col = jax.lax.broadcasted_iota(jnp.int32, (M_, NP_), 0)
    grp = jax.lax.broadcasted_iota(jnp.int32, (M_, NP_), 1)
    G = jnp.where(col // NK_ == grp, 1.0, 0.0).astype(jnp.float32)
    ps = jax.lax.dot_general(
        E, G, (((1,), (0,)), ((), ())),
        preferred_element_type=jnp.float32)
    pred_ref[...] = ps / jnp.sum(ps, axis=1, keepdims=True)


@functools.partial(jax.jit, static_argnames=())
def kernel(q, P):
    B = q.shape[0]
    nb = B // BR_
    pred, D, E = pl.pallas_call(
        _body,
        grid=(nb,),
        in_specs=[
            pl.BlockSpec((BR_, EMBED_), lambda i: (i, 0)),
            pl.BlockSpec((M_, EMBED_), lambda i: (0, 0)),
        ],
        out_specs=[
            pl.BlockSpec((BR_, NP_), lambda i: (i, 0)),
            pl.BlockSpec((BR_, M_), lambda i: (i, 0)),
            pl.BlockSpec((BR_, M_), lambda i: (i, 0)),
        ],
        out_shape=[
            jax.ShapeDtypeStruct((B, NP_), jnp.float32),
            jax.ShapeDtypeStruct((B, M_), jnp.float32),
            jax.ShapeDtypeStruct((B, M_), jnp.float32),
        ],
        compiler_params=pltpu.CompilerParams(
            dimension_semantics=("parallel",)),
    )(q, P)
    return (pred, D, E.reshape(B, NP_, NK_))
